# baseline (speedup 1.0000x reference)
import jax
import jax.numpy as jnp
from jax import lax
from jax.experimental import pallas as pl
from jax.experimental.pallas import tpu as pltpu
from jax.experimental.pallas import tpu_sc as plsc

NC = 2
NS = 16
NW = NC * NS
L = 16

ROWS = 16384
COLS = 1000
M = 200
MPAD = 208
NMV = MPAD // L
RPW = ROWS // NW
R = 32
NCHUNK = RPW // R


def _body(x_hbm, mask_hbm, out_hbm,
          mask_v, xv0, xv1, ov0, ov1, si0, si1, so0, so1):
    wid = lax.axis_index("s") * NC + lax.axis_index("c")
    base = wid * RPW

    pltpu.sync_copy(mask_hbm, mask_v)

    xvs = (xv0, xv1)
    ovs = (ov0, ov1)
    sis = (si0, si1)
    sos = (so0, so1)

    def start_in(g):
        b = g % 2
        return pltpu.make_async_copy(
            x_hbm.at[pl.ds(base + g * R, R)], xvs[b], sis[b])

    def start_out(g):
        b = g % 2
        return pltpu.make_async_copy(
            ovs[b], out_hbm.at[pl.ds(base + g * R, R)], sos[b])

    in_h = [None] * NCHUNK
    out_h = [None] * NCHUNK

    in_h[0] = start_in(0)
    in_h[0].start()

    for g in range(NCHUNK):
        b = g % 2
        if g + 1 < NCHUNK:
            in_h[g + 1] = start_in(g + 1)
            in_h[g + 1].start()
        in_h[g].wait()
        if g >= 2:
            out_h[g - 2].wait()

        xv, ov = xvs[b], ovs[b]

        def row(r, carry):
            rsplat = jnp.full((L,), 0, jnp.int32) + r
            for m in range(NMV):
                idx = mask_v[pl.ds(m * L, L)]
                vals = plsc.load_gather(xv, [rsplat, idx])
                if (m + 1) * L <= M:
                    ov[r, pl.ds(m * L, L)] = vals
                else:
                    cidx = m * L + lax.iota(jnp.int32, L)
                    plsc.store_scatter(ov, [rsplat, cidx], vals,
                                       mask=cidx < M)
            return carry

        lax.fori_loop(0, R, row, 0)

        out_h[g] = start_out(g)
        out_h[g].start()

    out_h[NCHUNK - 2].wait()
    out_h[NCHUNK - 1].wait()


def kernel(x, mask):
    mask_padded = jnp.concatenate(
        [mask, jnp.zeros((MPAD - M,), jnp.int32)])
    f = pl.kernel(
        _body,
        out_type=jax.ShapeDtypeStruct((ROWS, M), jnp.float32),
        mesh=plsc.VectorSubcoreMesh(core_axis_name="c", subcore_axis_name="s"),
        compiler_params=pltpu.CompilerParams(
            needs_layout_passes=False, use_tc_tiling_on_sc=False),
        scratch_types=[
            pltpu.VMEM((MPAD,), jnp.int32),
            pltpu.VMEM((R, COLS), jnp.float32),
            pltpu.VMEM((R, COLS), jnp.float32),
            pltpu.VMEM((R, M), jnp.float32),
            pltpu.VMEM((R, M), jnp.float32),
            pltpu.SemaphoreType.DMA,
            pltpu.SemaphoreType.DMA,
            pltpu.SemaphoreType.DMA,
            pltpu.SemaphoreType.DMA,
        ],
    )
    return f(x, mask_padded)


# speedup vs baseline: 1.4988x; 1.4988x over previous
import jax
import jax.numpy as jnp
from jax import lax
from jax.experimental import pallas as pl
from jax.experimental.pallas import tpu as pltpu
from jax.experimental.pallas import tpu_sc as plsc

NC = 2
NS = 16
NW = NC * NS
L = 16

ROWS = 16384
COLS = 1000
M = 200
MPAD = 208
NMV = MPAD // L
RPW = ROWS // NW
R = 32
NCHUNK = RPW // R


def _body(x_hbm, mask_hbm, out_hbm,
          mask_v, xv0, xv1, ov0, ov1, si0, si1, so0, so1):
    wid = lax.axis_index("s") * NC + lax.axis_index("c")
    base = wid * RPW

    pltpu.sync_copy(mask_hbm, mask_v)

    xvs = (xv0, xv1)
    ovs = (ov0, ov1)
    sis = (si0, si1)
    sos = (so0, so1)

    def start_in(g):
        b = g % 2
        return pltpu.make_async_copy(
            x_hbm.at[pl.ds(base + g * R, R)], xvs[b], sis[b])

    def start_out(g):
        b = g % 2
        return pltpu.make_async_copy(
            ovs[b], out_hbm.at[pl.ds(base + g * R, R)], sos[b])

    in_h = [None] * NCHUNK
    out_h = [None] * NCHUNK

    in_h[0] = start_in(0)
    in_h[0].start()

    for g in range(NCHUNK):
        b = g % 2
        if g + 1 < NCHUNK:
            in_h[g + 1] = start_in(g + 1)
            in_h[g + 1].start()
        in_h[g].wait()
        if g >= 2:
            out_h[g - 2].wait()

        xv, ov = xvs[b], ovs[b]

        def row(r, carry):
            rsplat = jnp.full((L,), 0, jnp.int32) + r
            for m in range(NMV):
                idx = mask_v[pl.ds(m * L, L)]
                vals = plsc.load_gather(xv, [rsplat, idx])
                if (m + 1) * L <= M:
                    ov[r, pl.ds(m * L, L)] = vals
                else:
                    cidx = m * L + lax.iota(jnp.int32, L)
                    plsc.store_scatter(ov, [rsplat, cidx], vals,
                                       mask=cidx < M)
            return carry

        lax.fori_loop(0, R, row, 0)

        out_h[g] = start_out(g)
        out_h[g].start()

    out_h[NCHUNK - 2].wait()
    out_h[NCHUNK - 1].wait()


def kernel(x, mask):
    mask_padded = jnp.concatenate(
        [mask, jnp.zeros((MPAD - M,), jnp.int32)])
    f = pl.kernel(
        _body,
        out_type=jax.ShapeDtypeStruct((ROWS, M), jnp.float32),
        mesh=plsc.VectorSubcoreMesh(core_axis_name="c", subcore_axis_name="s"),
        compiler_params=pltpu.CompilerParams(needs_layout_passes=False),
        scratch_types=[
            pltpu.VMEM((MPAD,), jnp.int32),
            pltpu.VMEM((R, COLS), jnp.float32),
            pltpu.VMEM((R, COLS), jnp.float32),
            pltpu.VMEM((R, M), jnp.float32),
            pltpu.VMEM((R, M), jnp.float32),
            pltpu.SemaphoreType.DMA,
            pltpu.SemaphoreType.DMA,
            pltpu.SemaphoreType.DMA,
            pltpu.SemaphoreType.DMA,
        ],
    )
    return f(x, mask_padded)
